# baseline (speedup 1.0000x reference)
import functools

import jax
import jax.numpy as jnp
from jax import lax
from jax.experimental import pallas as pl
from jax.experimental.pallas import tpu as pltpu
from jax.experimental.pallas import tpu_sc as plsc

N = 10000
D = 256
E = 160000

NC = 2
NS = 16
NW = NC * NS
OWN = 320
NPAD = NW * OWN
TRASH = OWN
ACC_ROWS = 328

G = 64
SUP = 1024
SCAN = 1280
NSCAN = E // SCAN
FB = 4096
LBUF = 8192
E_CAP = E + 2 * FB


def _wid():
    return lax.axis_index("s") * NC + lax.axis_index("c")


def _vgather(v, idx):
    return lax.gather(
        v, idx[:, None],
        dimension_numbers=lax.GatherDimensionNumbers(
            offset_dims=(), collapsed_slice_dims=(0,), start_index_map=(0,)),
        slice_sizes=(1,),
        mode=lax.GatherScatterMode.PROMISE_IN_BOUNDS)


def _prep_body(src_hbm, dst_hbm, ksrc_hbm, kloc_hbm, cnt_hbm,
               sbuf, dbuf, ksl, kll, cbuf, semA, semB):
    w = _wid()
    lo = w * OWN
    base = w * E_CAP

    lane15 = jnp.full((16,), 15, jnp.int32)

    def load_chunk(c):
        cm = pl.multiple_of(c * SCAN, 8)

        @pl.when(c % 2 == 0)
        def _():
            pltpu.async_copy(src_hbm.at[pl.ds(cm, SCAN)], sbuf.at[0], semA)
            pltpu.async_copy(dst_hbm.at[pl.ds(cm, SCAN)], dbuf.at[0], semA)

        @pl.when(c % 2 == 1)
        def _():
            pltpu.async_copy(src_hbm.at[pl.ds(cm, SCAN)], sbuf.at[1], semB)
            pltpu.async_copy(dst_hbm.at[pl.ds(cm, SCAN)], dbuf.at[1], semB)

    def wait_chunk(c):
        cm = pl.multiple_of(c * SCAN, 8)

        @pl.when(c % 2 == 0)
        def _():
            pltpu.make_async_copy(src_hbm.at[pl.ds(cm, SCAN)], sbuf.at[0], semA).wait()
            pltpu.make_async_copy(dst_hbm.at[pl.ds(cm, SCAN)], dbuf.at[0], semA).wait()

        @pl.when(c % 2 == 1)
        def _():
            pltpu.make_async_copy(src_hbm.at[pl.ds(cm, SCAN)], sbuf.at[1], semB).wait()
            pltpu.make_async_copy(dst_hbm.at[pl.ds(cm, SCAN)], dbuf.at[1], semB).wait()

    load_chunk(0)

    def scan_chunk(c, carry):
        cnt, off = carry
        cnt_s = jnp.broadcast_to(cnt, (16,)).astype(jnp.int32)

        @pl.when(c + 1 < NSCAN)
        def _():
            load_chunk(c + 1)

        wait_chunk(c)
        p = c % 2
        for k in range(SCAN // 16):
            s = sbuf[p, pl.ds(16 * k, 16)]
            d = dbuf[p, pl.ds(16 * k, 16)]
            lr = d - lo
            m = (lr >= 0) & (lr < OWN)
            pc = plsc.cumsum(jnp.where(m, 1, 0))
            pos = cnt_s + pc - 1
            plsc.store_scatter(ksl, [pos], s, mask=m)
            plsc.store_scatter(kll, [pos], lr, mask=m)
            cnt_s = cnt_s + _vgather(pc, lane15)
        cnt = jnp.max(cnt_s)
        flushed = cnt >= FB

        @pl.when(flushed)
        def _():
            fo = pl.multiple_of(base + off, 8)
            pltpu.sync_copy(ksl.at[pl.ds(0, FB)],
                            ksrc_hbm.at[pl.ds(fo, FB)])
            pltpu.sync_copy(kll.at[pl.ds(0, FB)],
                            kloc_hbm.at[pl.ds(fo, FB)])
            for k in range(SCAN // 16):
                ksl[pl.ds(16 * k, 16)] = ksl[pl.ds(FB + 16 * k, 16)]
                kll[pl.ds(16 * k, 16)] = kll[pl.ds(FB + 16 * k, 16)]

        cnt = jnp.where(flushed, cnt - FB, cnt)
        off = jnp.where(flushed, off + FB, off)
        return cnt, off

    cnt, off = lax.fori_loop(0, NSCAN, scan_chunk,
                             (jnp.int32(0), jnp.int32(0)))

    for k in range(G // 16):
        ksl[pl.ds(cnt + 16 * k, 16)] = jnp.zeros((16,), jnp.int32)
        kll[pl.ds(cnt + 16 * k, 16)] = jnp.full((16,), TRASH, jnp.int32)
    cnt_p = cnt - (cnt % G) + G

    nbf = (cnt_p + FB - 1) >> 12

    def final_flush(k, carry):
        fo = pl.multiple_of(base + off + k * FB, 8)
        pltpu.sync_copy(ksl.at[pl.ds(k * FB, FB)],
                        ksrc_hbm.at[pl.ds(fo, FB)])
        pltpu.sync_copy(kll.at[pl.ds(k * FB, FB)],
                        kloc_hbm.at[pl.ds(fo, FB)])
        return carry

    lax.fori_loop(0, nbf, final_flush, 0)

    total = off + cnt_p
    cbuf[...] = jnp.broadcast_to(total, (16,)).astype(jnp.int32)
    pltpu.sync_copy(cbuf, cnt_hbm.at[pl.ds(pl.multiple_of(w * 16, 8), 16)])


_prep = pl.kernel(
    _prep_body,
    out_type=(
        jax.ShapeDtypeStruct((NW * E_CAP,), jnp.int32),
        jax.ShapeDtypeStruct((NW * E_CAP,), jnp.int32),
        jax.ShapeDtypeStruct((NW * 16,), jnp.int32),
    ),
    mesh=plsc.VectorSubcoreMesh(core_axis_name="c", subcore_axis_name="s"),
    compiler_params=pltpu.CompilerParams(needs_layout_passes=False),
    scratch_types=[
        pltpu.VMEM((2, SCAN), jnp.int32),
        pltpu.VMEM((2, SCAN), jnp.int32),
        pltpu.VMEM((LBUF,), jnp.int32),
        pltpu.VMEM((LBUF,), jnp.int32),
        pltpu.VMEM((16,), jnp.int32),
        pltpu.SemaphoreType.DMA,
        pltpu.SemaphoreType.DMA,
    ],
)


def _seg_body(h_hbm, ksrc_hbm, kloc_hbm, cnt_hbm, out_hbm,
              acc, sbuf2, lbuf2, rows, cbuf, semA, semB):
    w = _wid()
    base = w * E_CAP

    def zero_row(r, carry):
        for j in range(D // 16):
            acc[r, pl.ds(16 * j, 16)] = jnp.zeros((16,), jnp.float32)
        return carry

    lax.fori_loop(0, ACC_ROWS, zero_row, 0)

    pltpu.sync_copy(cnt_hbm.at[pl.ds(pl.multiple_of(w * 16, 8), 16)], cbuf)
    cnt = jnp.max(cbuf[...])
    nb = cnt >> 6
    nsc = (nb + 15) >> 4

    lane = lax.iota(jnp.int32, 16)
    cols = [lane + 16 * j for j in range(D // 16)]

    def gather_start(k):
        idx = sbuf2.at[pl.ds(k * G, G)]

        @pl.when(k % 2 == 0)
        def _():
            pltpu.async_copy(h_hbm.at[idx], rows.at[0], semA)

        @pl.when(k % 2 == 1)
        def _():
            pltpu.async_copy(h_hbm.at[idx], rows.at[1], semB)

    def accum(k):
        idx = sbuf2.at[pl.ds(k * G, G)]

        def do(p, sem):
            pltpu.make_async_copy(h_hbm.at[idx], rows.at[p], sem).wait()

            def accum_group(gi, carry):
                lv = lbuf2[pl.ds(k * G + gi * 16, 16)]
                svs = [_vgather(lv, jnp.full((16,), e, jnp.int32))
                       for e in range(16)]
                for j in range(D // 16):
                    for e in range(16):
                        plsc.addupdate_scatter(
                            acc, [svs[e], cols[j]],
                            rows[p, gi * 16 + e, pl.ds(16 * j, 16)])
                return carry

            lax.fori_loop(0, G // 16, accum_group, 0)

        @pl.when(k % 2 == 0)
        def _():
            do(0, semA)

        @pl.when(k % 2 == 1)
        def _():
            do(1, semB)

    def superchunk(sc, carry):
        e0 = pl.multiple_of(base + sc * SUP, 8)
        pltpu.sync_copy(ksrc_hbm.at[pl.ds(e0, SUP)], sbuf2)
        pltpu.sync_copy(kloc_hbm.at[pl.ds(e0, SUP)], lbuf2)
        m = jnp.minimum(16, nb - sc * 16)

        gather_start(0)

        def inner(k, carry):
            gather_start(k)
            accum(k - 1)
            return carry

        lax.fori_loop(1, m, inner, 0)
        accum(m - 1)
        return carry

    lax.fori_loop(0, nsc, superchunk, 0)

    pltpu.sync_copy(acc.at[pl.ds(0, OWN)], out_hbm.at[pl.ds(pl.multiple_of(w * OWN, 8), OWN)])


_seg = pl.kernel(
    _seg_body,
    out_type=jax.ShapeDtypeStruct((NPAD, D), jnp.float32),
    mesh=plsc.VectorSubcoreMesh(core_axis_name="c", subcore_axis_name="s"),
    compiler_params=pltpu.CompilerParams(needs_layout_passes=False),
    scratch_types=[
        pltpu.VMEM((ACC_ROWS, D), jnp.float32),
        pltpu.VMEM((SUP,), jnp.int32),
        pltpu.VMEM((SUP,), jnp.int32),
        pltpu.VMEM((2, G, D), jnp.float32),
        pltpu.VMEM((16,), jnp.int32),
        pltpu.SemaphoreType.DMA,
        pltpu.SemaphoreType.DMA,
    ],
)


def _bn(y, g, b):
    m = jnp.mean(y, axis=0, keepdims=True)
    v = jnp.mean((y - m) ** 2, axis=0, keepdims=True)
    return g * (y - m) / jnp.sqrt(v + 1e-5) + b


def _mlp_body(final, *refs):
    if final:
        (x_ref, agg_ref, w1t, b1, g1, be1, w2t, b2, g2, be2, go, beo,
         wfct, bfc, out_ref) = refs
    else:
        (x_ref, agg_ref, w1t, b1, g1, be1, w2t, b2, g2, be2, go, beo,
         out_ref) = refs
    u = x_ref[...] + agg_ref[...]
    y = jnp.dot(u, w1t[...], preferred_element_type=jnp.float32) + b1[...]
    y = jnp.maximum(_bn(y, g1[...], be1[...]), 0.0)
    y = jnp.dot(y, w2t[...], preferred_element_type=jnp.float32) + b2[...]
    y = _bn(y, g2[...], be2[...])
    y = jnp.maximum(_bn(y, go[...], beo[...]), 0.0)
    if final:
        y = jnp.dot(y, wfct[...], preferred_element_type=jnp.float32) + bfc[...]
    out_ref[...] = y


def _mlp_call(final):
    return pl.pallas_call(
        functools.partial(_mlp_body, final),
        out_shape=jax.ShapeDtypeStruct((N, D), jnp.float32),
    )


def kernel(x, edge_index, params):
    src = edge_index[0]
    dst = edge_index[1]
    ksrc, kloc, cnts = _prep(src, dst)
    h = x
    for i in range(3):
        agg = _seg(h, ksrc, kloc, cnts)[:N]
        args = [h, agg,
                params[f"W1_{i}"].T, params[f"b1_{i}"].reshape(1, D),
                params[f"g1_{i}"].reshape(1, D), params[f"be1_{i}"].reshape(1, D),
                params[f"W2_{i}"].T, params[f"b2_{i}"].reshape(1, D),
                params[f"g2_{i}"].reshape(1, D), params[f"be2_{i}"].reshape(1, D),
                params[f"go_{i}"].reshape(1, D), params[f"beo_{i}"].reshape(1, D)]
        final = i == 2
        if final:
            args += [params["Wfc"].T, params["bfc"].reshape(1, D)]
        h = _mlp_call(final)(*args)
    return h


# speedup vs baseline: 1.0194x; 1.0194x over previous
import functools

import jax
import jax.numpy as jnp
from jax import lax
from jax.experimental import pallas as pl
from jax.experimental.pallas import tpu as pltpu
from jax.experimental.pallas import tpu_sc as plsc

N = 10000
D = 256
E = 160000

NC = 2
NS = 16
NW = NC * NS
OWN = 320
NPAD = NW * OWN
TRASH = OWN
ACC_ROWS = 328

G = 64
SUP = 2048
SCAN = 1280
NSCAN = E // SCAN
FB = 4096
LBUF = 8192
E_CAP = E + 2 * FB


def _wid():
    return lax.axis_index("s") * NC + lax.axis_index("c")


def _vgather(v, idx):
    return lax.gather(
        v, idx[:, None],
        dimension_numbers=lax.GatherDimensionNumbers(
            offset_dims=(), collapsed_slice_dims=(0,), start_index_map=(0,)),
        slice_sizes=(1,),
        mode=lax.GatherScatterMode.PROMISE_IN_BOUNDS)


def _prep_body(src_hbm, dst_hbm, ksrc_hbm, kloc_hbm, cnt_hbm,
               sbuf, dbuf, ksl, kll, cbuf, semA, semB):
    w = _wid()
    lo = w * OWN
    base = w * E_CAP

    lane15 = jnp.full((16,), 15, jnp.int32)

    def load_chunk(c):
        cm = pl.multiple_of(c * SCAN, 8)

        @pl.when(c % 2 == 0)
        def _():
            pltpu.async_copy(src_hbm.at[pl.ds(cm, SCAN)], sbuf.at[0], semA)
            pltpu.async_copy(dst_hbm.at[pl.ds(cm, SCAN)], dbuf.at[0], semA)

        @pl.when(c % 2 == 1)
        def _():
            pltpu.async_copy(src_hbm.at[pl.ds(cm, SCAN)], sbuf.at[1], semB)
            pltpu.async_copy(dst_hbm.at[pl.ds(cm, SCAN)], dbuf.at[1], semB)

    def wait_chunk(c):
        cm = pl.multiple_of(c * SCAN, 8)

        @pl.when(c % 2 == 0)
        def _():
            pltpu.make_async_copy(src_hbm.at[pl.ds(cm, SCAN)], sbuf.at[0], semA).wait()
            pltpu.make_async_copy(dst_hbm.at[pl.ds(cm, SCAN)], dbuf.at[0], semA).wait()

        @pl.when(c % 2 == 1)
        def _():
            pltpu.make_async_copy(src_hbm.at[pl.ds(cm, SCAN)], sbuf.at[1], semB).wait()
            pltpu.make_async_copy(dst_hbm.at[pl.ds(cm, SCAN)], dbuf.at[1], semB).wait()

    load_chunk(0)

    def scan_chunk(c, carry):
        cnt, off = carry
        cnt_s = jnp.broadcast_to(cnt, (16,)).astype(jnp.int32)

        @pl.when(c + 1 < NSCAN)
        def _():
            load_chunk(c + 1)

        wait_chunk(c)
        p = c % 2
        for k in range(SCAN // 16):
            s = sbuf[p, pl.ds(16 * k, 16)]
            d = dbuf[p, pl.ds(16 * k, 16)]
            lr = d - lo
            m = (lr >= 0) & (lr < OWN)
            pc = plsc.cumsum(jnp.where(m, 1, 0))
            pos = cnt_s + pc - 1
            plsc.store_scatter(ksl, [pos], s, mask=m)
            plsc.store_scatter(kll, [pos], lr, mask=m)
            cnt_s = cnt_s + _vgather(pc, lane15)
        cnt = jnp.max(cnt_s)
        flushed = cnt >= FB

        @pl.when(flushed)
        def _():
            fo = pl.multiple_of(base + off, 8)
            pltpu.sync_copy(ksl.at[pl.ds(0, FB)],
                            ksrc_hbm.at[pl.ds(fo, FB)])
            pltpu.sync_copy(kll.at[pl.ds(0, FB)],
                            kloc_hbm.at[pl.ds(fo, FB)])
            for k in range(SCAN // 16):
                ksl[pl.ds(16 * k, 16)] = ksl[pl.ds(FB + 16 * k, 16)]
                kll[pl.ds(16 * k, 16)] = kll[pl.ds(FB + 16 * k, 16)]

        cnt = jnp.where(flushed, cnt - FB, cnt)
        off = jnp.where(flushed, off + FB, off)
        return cnt, off

    cnt, off = lax.fori_loop(0, NSCAN, scan_chunk,
                             (jnp.int32(0), jnp.int32(0)))

    for k in range(G // 16):
        ksl[pl.ds(cnt + 16 * k, 16)] = jnp.zeros((16,), jnp.int32)
        kll[pl.ds(cnt + 16 * k, 16)] = jnp.full((16,), TRASH, jnp.int32)
    cnt_p = cnt - (cnt % G) + G

    nbf = (cnt_p + FB - 1) >> 12

    def final_flush(k, carry):
        fo = pl.multiple_of(base + off + k * FB, 8)
        pltpu.sync_copy(ksl.at[pl.ds(k * FB, FB)],
                        ksrc_hbm.at[pl.ds(fo, FB)])
        pltpu.sync_copy(kll.at[pl.ds(k * FB, FB)],
                        kloc_hbm.at[pl.ds(fo, FB)])
        return carry

    lax.fori_loop(0, nbf, final_flush, 0)

    total = off + cnt_p
    cbuf[...] = jnp.broadcast_to(total, (16,)).astype(jnp.int32)
    pltpu.sync_copy(cbuf, cnt_hbm.at[pl.ds(pl.multiple_of(w * 16, 8), 16)])


_prep = pl.kernel(
    _prep_body,
    out_type=(
        jax.ShapeDtypeStruct((NW * E_CAP,), jnp.int32),
        jax.ShapeDtypeStruct((NW * E_CAP,), jnp.int32),
        jax.ShapeDtypeStruct((NW * 16,), jnp.int32),
    ),
    mesh=plsc.VectorSubcoreMesh(core_axis_name="c", subcore_axis_name="s"),
    compiler_params=pltpu.CompilerParams(needs_layout_passes=False),
    scratch_types=[
        pltpu.VMEM((2, SCAN), jnp.int32),
        pltpu.VMEM((2, SCAN), jnp.int32),
        pltpu.VMEM((LBUF,), jnp.int32),
        pltpu.VMEM((LBUF,), jnp.int32),
        pltpu.VMEM((16,), jnp.int32),
        pltpu.SemaphoreType.DMA,
        pltpu.SemaphoreType.DMA,
    ],
)


def _seg_body(h_hbm, ksrc_hbm, kloc_hbm, cnt_hbm, out_hbm,
              acc, sbuf2, lbuf2, rows, cbuf, semA, semB):
    w = _wid()
    base = w * E_CAP

    def zero_row(r, carry):
        for j in range(D // 16):
            acc[r, pl.ds(16 * j, 16)] = jnp.zeros((16,), jnp.float32)
        return carry

    lax.fori_loop(0, ACC_ROWS, zero_row, 0)

    pltpu.sync_copy(cnt_hbm.at[pl.ds(pl.multiple_of(w * 16, 8), 16)], cbuf)
    cnt = jnp.max(cbuf[...])
    nb = cnt >> 6
    nsc = (nb + 31) >> 5

    lane = lax.iota(jnp.int32, 16)
    cols = [lane + 16 * j for j in range(D // 16)]

    def gather_start(k):
        idx = sbuf2.at[pl.ds(k * G, G)]

        @pl.when(k % 2 == 0)
        def _():
            pltpu.async_copy(h_hbm.at[idx], rows.at[0], semA)

        @pl.when(k % 2 == 1)
        def _():
            pltpu.async_copy(h_hbm.at[idx], rows.at[1], semB)

    def accum(k):
        idx = sbuf2.at[pl.ds(k * G, G)]

        @pl.when(k % 2 == 0)
        def _():
            pltpu.make_async_copy(h_hbm.at[idx], rows.at[0], semA).wait()

        @pl.when(k % 2 == 1)
        def _():
            pltpu.make_async_copy(h_hbm.at[idx], rows.at[1], semB).wait()

        p = k % 2

        def accum_group(gi, carry):
            lv = lbuf2[pl.ds(k * G + gi * 16, 16)]
            for e in range(16):
                sv = _vgather(lv, jnp.full((16,), e, jnp.int32))
                i = gi * 16 + e
                for j in range(D // 16):
                    plsc.addupdate_scatter(
                        acc, [sv, cols[j]], rows[p, i, pl.ds(16 * j, 16)])
            return carry

        lax.fori_loop(0, G // 16, accum_group, 0)

    def superchunk(sc, carry):
        e0 = pl.multiple_of(base + sc * SUP, 8)
        pltpu.sync_copy(ksrc_hbm.at[pl.ds(e0, SUP)], sbuf2)
        pltpu.sync_copy(kloc_hbm.at[pl.ds(e0, SUP)], lbuf2)
        m = jnp.minimum(32, nb - sc * 32)

        gather_start(0)

        def inner(k, carry):
            gather_start(k)
            accum(k - 1)
            return carry

        lax.fori_loop(1, m, inner, 0)
        accum(m - 1)
        return carry

    lax.fori_loop(0, nsc, superchunk, 0)

    pltpu.sync_copy(acc.at[pl.ds(0, OWN)], out_hbm.at[pl.ds(pl.multiple_of(w * OWN, 8), OWN)])


_seg = pl.kernel(
    _seg_body,
    out_type=jax.ShapeDtypeStruct((NPAD, D), jnp.float32),
    mesh=plsc.VectorSubcoreMesh(core_axis_name="c", subcore_axis_name="s"),
    compiler_params=pltpu.CompilerParams(needs_layout_passes=False),
    scratch_types=[
        pltpu.VMEM((ACC_ROWS, D), jnp.float32),
        pltpu.VMEM((SUP,), jnp.int32),
        pltpu.VMEM((SUP,), jnp.int32),
        pltpu.VMEM((2, G, D), jnp.float32),
        pltpu.VMEM((16,), jnp.int32),
        pltpu.SemaphoreType.DMA,
        pltpu.SemaphoreType.DMA,
    ],
)


def _bn(y, g, b):
    m = jnp.mean(y, axis=0, keepdims=True)
    v = jnp.mean((y - m) ** 2, axis=0, keepdims=True)
    return g * (y - m) / jnp.sqrt(v + 1e-5) + b


def _mlp_body(final, *refs):
    if final:
        (x_ref, agg_ref, w1t, b1, g1, be1, w2t, b2, g2, be2, go, beo,
         wfct, bfc, out_ref) = refs
    else:
        (x_ref, agg_ref, w1t, b1, g1, be1, w2t, b2, g2, be2, go, beo,
         out_ref) = refs
    u = x_ref[...] + agg_ref[...]
    y = jnp.dot(u, w1t[...], preferred_element_type=jnp.float32) + b1[...]
    y = jnp.maximum(_bn(y, g1[...], be1[...]), 0.0)
    y = jnp.dot(y, w2t[...], preferred_element_type=jnp.float32) + b2[...]
    y = _bn(y, g2[...], be2[...])
    y = jnp.maximum(_bn(y, go[...], beo[...]), 0.0)
    if final:
        y = jnp.dot(y, wfct[...], preferred_element_type=jnp.float32) + bfc[...]
    out_ref[...] = y


def _mlp_call(final):
    return pl.pallas_call(
        functools.partial(_mlp_body, final),
        out_shape=jax.ShapeDtypeStruct((N, D), jnp.float32),
    )


def kernel(x, edge_index, params):
    src = edge_index[0]
    dst = edge_index[1]
    ksrc, kloc, cnts = _prep(src, dst)
    h = x
    for i in range(3):
        agg = _seg(h, ksrc, kloc, cnts)[:N]
        args = [h, agg,
                params[f"W1_{i}"].T, params[f"b1_{i}"].reshape(1, D),
                params[f"g1_{i}"].reshape(1, D), params[f"be1_{i}"].reshape(1, D),
                params[f"W2_{i}"].T, params[f"b2_{i}"].reshape(1, D),
                params[f"g2_{i}"].reshape(1, D), params[f"be2_{i}"].reshape(1, D),
                params[f"go_{i}"].reshape(1, D), params[f"beo_{i}"].reshape(1, D)]
        final = i == 2
        if final:
            args += [params["Wfc"].T, params["bfc"].reshape(1, D)]
        h = _mlp_call(final)(*args)
    return h


# speedup vs baseline: 1.0475x; 1.0275x over previous
import functools

import jax
import jax.numpy as jnp
from jax import lax
from jax.experimental import pallas as pl
from jax.experimental.pallas import tpu as pltpu
from jax.experimental.pallas import tpu_sc as plsc

N = 10000
D = 256
E = 160000

NC = 2
NS = 16
NW = NC * NS
OWN = 320
NPAD = NW * OWN
TRASH = OWN
ACC_ROWS = 328

G = 32
SUP = 2048
SCAN = 1280
NSCAN = E // SCAN
FB = 4096
LBUF = 8192
E_CAP = E + 2 * FB


def _wid():
    return lax.axis_index("s") * NC + lax.axis_index("c")


def _vgather(v, idx):
    return lax.gather(
        v, idx[:, None],
        dimension_numbers=lax.GatherDimensionNumbers(
            offset_dims=(), collapsed_slice_dims=(0,), start_index_map=(0,)),
        slice_sizes=(1,),
        mode=lax.GatherScatterMode.PROMISE_IN_BOUNDS)


def _prep_body(src_hbm, dst_hbm, ksrc_hbm, kloc_hbm, cnt_hbm,
               sbuf, dbuf, ksl, kll, cbuf, semA, semB):
    w = _wid()
    lo = w * OWN
    base = w * E_CAP

    lane15 = jnp.full((16,), 15, jnp.int32)

    def load_chunk(c):
        cm = pl.multiple_of(c * SCAN, 8)

        @pl.when(c % 2 == 0)
        def _():
            pltpu.async_copy(src_hbm.at[pl.ds(cm, SCAN)], sbuf.at[0], semA)
            pltpu.async_copy(dst_hbm.at[pl.ds(cm, SCAN)], dbuf.at[0], semA)

        @pl.when(c % 2 == 1)
        def _():
            pltpu.async_copy(src_hbm.at[pl.ds(cm, SCAN)], sbuf.at[1], semB)
            pltpu.async_copy(dst_hbm.at[pl.ds(cm, SCAN)], dbuf.at[1], semB)

    def wait_chunk(c):
        cm = pl.multiple_of(c * SCAN, 8)

        @pl.when(c % 2 == 0)
        def _():
            pltpu.make_async_copy(src_hbm.at[pl.ds(cm, SCAN)], sbuf.at[0], semA).wait()
            pltpu.make_async_copy(dst_hbm.at[pl.ds(cm, SCAN)], dbuf.at[0], semA).wait()

        @pl.when(c % 2 == 1)
        def _():
            pltpu.make_async_copy(src_hbm.at[pl.ds(cm, SCAN)], sbuf.at[1], semB).wait()
            pltpu.make_async_copy(dst_hbm.at[pl.ds(cm, SCAN)], dbuf.at[1], semB).wait()

    load_chunk(0)

    def scan_chunk(c, carry):
        cnt, off = carry
        cnt_s = jnp.broadcast_to(cnt, (16,)).astype(jnp.int32)

        @pl.when(c + 1 < NSCAN)
        def _():
            load_chunk(c + 1)

        wait_chunk(c)
        p = c % 2
        for k in range(SCAN // 16):
            s = sbuf[p, pl.ds(16 * k, 16)]
            d = dbuf[p, pl.ds(16 * k, 16)]
            lr = d - lo
            m = (lr >= 0) & (lr < OWN)
            pc = plsc.cumsum(jnp.where(m, 1, 0))
            pos = cnt_s + pc - 1
            plsc.store_scatter(ksl, [pos], s, mask=m)
            plsc.store_scatter(kll, [pos], lr, mask=m)
            cnt_s = cnt_s + _vgather(pc, lane15)
        cnt = jnp.max(cnt_s)
        flushed = cnt >= FB

        @pl.when(flushed)
        def _():
            fo = pl.multiple_of(base + off, 8)
            pltpu.sync_copy(ksl.at[pl.ds(0, FB)],
                            ksrc_hbm.at[pl.ds(fo, FB)])
            pltpu.sync_copy(kll.at[pl.ds(0, FB)],
                            kloc_hbm.at[pl.ds(fo, FB)])
            for k in range(SCAN // 16):
                ksl[pl.ds(16 * k, 16)] = ksl[pl.ds(FB + 16 * k, 16)]
                kll[pl.ds(16 * k, 16)] = kll[pl.ds(FB + 16 * k, 16)]

        cnt = jnp.where(flushed, cnt - FB, cnt)
        off = jnp.where(flushed, off + FB, off)
        return cnt, off

    cnt, off = lax.fori_loop(0, NSCAN, scan_chunk,
                             (jnp.int32(0), jnp.int32(0)))

    for k in range(G // 16):
        ksl[pl.ds(cnt + 16 * k, 16)] = jnp.zeros((16,), jnp.int32)
        kll[pl.ds(cnt + 16 * k, 16)] = jnp.full((16,), TRASH, jnp.int32)
    cnt_p = cnt - (cnt % G) + G

    nbf = (cnt_p + FB - 1) >> 12

    def final_flush(k, carry):
        fo = pl.multiple_of(base + off + k * FB, 8)
        pltpu.sync_copy(ksl.at[pl.ds(k * FB, FB)],
                        ksrc_hbm.at[pl.ds(fo, FB)])
        pltpu.sync_copy(kll.at[pl.ds(k * FB, FB)],
                        kloc_hbm.at[pl.ds(fo, FB)])
        return carry

    lax.fori_loop(0, nbf, final_flush, 0)

    total = off + cnt_p
    cbuf[...] = jnp.broadcast_to(total, (16,)).astype(jnp.int32)
    pltpu.sync_copy(cbuf, cnt_hbm.at[pl.ds(pl.multiple_of(w * 16, 8), 16)])


_prep = pl.kernel(
    _prep_body,
    out_type=(
        jax.ShapeDtypeStruct((NW * E_CAP,), jnp.int32),
        jax.ShapeDtypeStruct((NW * E_CAP,), jnp.int32),
        jax.ShapeDtypeStruct((NW * 16,), jnp.int32),
    ),
    mesh=plsc.VectorSubcoreMesh(core_axis_name="c", subcore_axis_name="s"),
    compiler_params=pltpu.CompilerParams(needs_layout_passes=False),
    scratch_types=[
        pltpu.VMEM((2, SCAN), jnp.int32),
        pltpu.VMEM((2, SCAN), jnp.int32),
        pltpu.VMEM((LBUF,), jnp.int32),
        pltpu.VMEM((LBUF,), jnp.int32),
        pltpu.VMEM((16,), jnp.int32),
        pltpu.SemaphoreType.DMA,
        pltpu.SemaphoreType.DMA,
    ],
)


def _seg_body(h_hbm, ksrc_hbm, kloc_hbm, cnt_hbm, out_hbm,
              acc, sbuf2, lbuf2, rows, cbuf, semA, semB, semC, semD):
    w = _wid()
    base = w * E_CAP
    sems = (semA, semB, semC, semD)

    def zero_row(r, carry):
        for j in range(D // 16):
            acc[r, pl.ds(16 * j, 16)] = jnp.zeros((16,), jnp.float32)
        return carry

    lax.fori_loop(0, ACC_ROWS, zero_row, 0)

    pltpu.sync_copy(cnt_hbm.at[pl.ds(pl.multiple_of(w * 16, 8), 16)], cbuf)
    cnt = jnp.max(cbuf[...])
    nb = cnt >> 5
    nsc = (nb + 63) >> 6

    lane = lax.iota(jnp.int32, 16)
    cols = [lane + 16 * j for j in range(D // 16)]

    def gather_start(k):
        idx = sbuf2.at[pl.ds(k * G, G)]
        for q, sm in enumerate(sems):
            @pl.when(k % 4 == q)
            def _(q=q, sm=sm):
                pltpu.async_copy(h_hbm.at[idx], rows.at[q], sm)

    def accum(k):
        idx = sbuf2.at[pl.ds(k * G, G)]
        for q, sm in enumerate(sems):
            @pl.when(k % 4 == q)
            def _(q=q, sm=sm):
                pltpu.make_async_copy(h_hbm.at[idx], rows.at[q], sm).wait()

        p = k % 4

        def accum_group(gi, carry):
            lv = lbuf2[pl.ds(k * G + gi * 16, 16)]
            for e in range(16):
                sv = _vgather(lv, jnp.full((16,), e, jnp.int32))
                i = gi * 16 + e
                for j in range(D // 16):
                    plsc.addupdate_scatter(
                        acc, [sv, cols[j]], rows[p, i, pl.ds(16 * j, 16)])
            return carry

        lax.fori_loop(0, G // 16, accum_group, 0)

    def superchunk(sc, carry):
        e0 = pl.multiple_of(base + sc * SUP, 8)
        pltpu.sync_copy(ksrc_hbm.at[pl.ds(e0, SUP)], sbuf2)
        pltpu.sync_copy(kloc_hbm.at[pl.ds(e0, SUP)], lbuf2)
        m = jnp.minimum(64, nb - sc * 64)

        gather_start(0)
        for t0 in (1, 2):
            @pl.when(t0 < m)
            def _(t0=t0):
                gather_start(t0)

        def inner(k, carry):
            gather_start(k)
            accum(k - 3)
            return carry

        lax.fori_loop(3, m, inner, 0)
        for t0 in range(3):
            jj = m - 3 + t0

            @pl.when(jj >= 0)
            def _(jj=jj):
                accum(jj)
        return carry

    lax.fori_loop(0, nsc, superchunk, 0)

    pltpu.sync_copy(acc.at[pl.ds(0, OWN)], out_hbm.at[pl.ds(pl.multiple_of(w * OWN, 8), OWN)])


_seg = pl.kernel(
    _seg_body,
    out_type=jax.ShapeDtypeStruct((NPAD, D), jnp.float32),
    mesh=plsc.VectorSubcoreMesh(core_axis_name="c", subcore_axis_name="s"),
    compiler_params=pltpu.CompilerParams(needs_layout_passes=False),
    scratch_types=[
        pltpu.VMEM((ACC_ROWS, D), jnp.float32),
        pltpu.VMEM((SUP,), jnp.int32),
        pltpu.VMEM((SUP,), jnp.int32),
        pltpu.VMEM((4, G, D), jnp.float32),
        pltpu.VMEM((16,), jnp.int32),
        pltpu.SemaphoreType.DMA,
        pltpu.SemaphoreType.DMA,
        pltpu.SemaphoreType.DMA,
        pltpu.SemaphoreType.DMA,
    ],
)


def _bn(y, g, b):
    m = jnp.mean(y, axis=0, keepdims=True)
    v = jnp.mean((y - m) ** 2, axis=0, keepdims=True)
    return g * (y - m) / jnp.sqrt(v + 1e-5) + b


def _mlp_body(final, *refs):
    if final:
        (x_ref, agg_ref, w1t, b1, g1, be1, w2t, b2, g2, be2, go, beo,
         wfct, bfc, out_ref) = refs
    else:
        (x_ref, agg_ref, w1t, b1, g1, be1, w2t, b2, g2, be2, go, beo,
         out_ref) = refs
    u = x_ref[...] + agg_ref[...]
    y = jnp.dot(u, w1t[...], preferred_element_type=jnp.float32) + b1[...]
    y = jnp.maximum(_bn(y, g1[...], be1[...]), 0.0)
    y = jnp.dot(y, w2t[...], preferred_element_type=jnp.float32) + b2[...]
    y = _bn(y, g2[...], be2[...])
    y = jnp.maximum(_bn(y, go[...], beo[...]), 0.0)
    if final:
        y = jnp.dot(y, wfct[...], preferred_element_type=jnp.float32) + bfc[...]
    out_ref[...] = y


def _mlp_call(final):
    return pl.pallas_call(
        functools.partial(_mlp_body, final),
        out_shape=jax.ShapeDtypeStruct((N, D), jnp.float32),
    )


def kernel(x, edge_index, params):
    src = edge_index[0]
    dst = edge_index[1]
    ksrc, kloc, cnts = _prep(src, dst)
    h = x
    for i in range(3):
        agg = _seg(h, ksrc, kloc, cnts)[:N]
        args = [h, agg,
                params[f"W1_{i}"].T, params[f"b1_{i}"].reshape(1, D),
                params[f"g1_{i}"].reshape(1, D), params[f"be1_{i}"].reshape(1, D),
                params[f"W2_{i}"].T, params[f"b2_{i}"].reshape(1, D),
                params[f"g2_{i}"].reshape(1, D), params[f"be2_{i}"].reshape(1, D),
                params[f"go_{i}"].reshape(1, D), params[f"beo_{i}"].reshape(1, D)]
        final = i == 2
        if final:
            args += [params["Wfc"].T, params["bfc"].reshape(1, D)]
        h = _mlp_call(final)(*args)
    return h
